# Initial kernel scaffold; baseline (speedup 1.0000x reference)
import jax
import jax.numpy as jnp
from jax.experimental import pallas as pl


def kernel(x, edge_index, batch_index, W0, b0, W1, b1, W2, b2, W3, b3, Wout, bout):
    raise NotImplementedError("write your pallas kernel here")


# speedup vs baseline: 12.1494x; 12.1494x over previous
import functools

import jax
import jax.numpy as jnp
from jax import lax
from jax.experimental import pallas as pl
from jax.experimental.pallas import tpu as pltpu
from jax.experimental.pallas import tpu_sc as plsc

N = 10000
D = 128
E = 320000
B = 64

NC = 2
NS = 16
NW = NC * NS
EPW = E // NW
CH = 128
NFULL = EPW // CH
TAIL = EPW - NFULL * CH
NP = 10240
RPT = NP // NS
ZR = 128

_mesh = plsc.VectorSubcoreMesh(core_axis_name="c", subcore_axis_name="s")


def _zero_fill(ref, rows, width):
    per_row = width // 16

    def body(k, _):
        i = k // per_row
        j = k % per_row
        ref[i, pl.ds(j * 16, 16)] = jnp.zeros((16,), jnp.float32)
        return 0

    lax.fori_loop(0, rows * per_row, body, 0)


def _deg_body(dst, degp, dst_v, dst_t, ones_v, zbuf, acc):
    cid = lax.axis_index("c")
    sid = lax.axis_index("s")
    wid = cid * NS + sid

    _zero_fill(zbuf, ZR, D)

    def ones_body(k, _):
        ones_v[k // 8, pl.ds((k % 8) * 16, 16)] = jnp.ones((16,), jnp.float32)
        return 0

    lax.fori_loop(0, CH * 8, ones_body, 0)
    for z in range(RPT // ZR):
        pltpu.sync_copy(zbuf, acc.at[pl.ds(sid * RPT + z * ZR, ZR)])
    plsc.subcore_barrier()

    ebase = wid * EPW

    def chunk(s, _):
        base = ebase + s * CH
        pltpu.sync_copy(dst.at[pl.ds(base, CH)], dst_v)
        pltpu.sync_copy(ones_v, acc.at[dst_v], add=True)
        return 0

    lax.fori_loop(0, NFULL, chunk, 0)
    base = ebase + NFULL * CH
    pltpu.sync_copy(dst.at[pl.ds(base, TAIL)], dst_t)
    pltpu.sync_copy(ones_v.at[pl.ds(0, TAIL)], acc.at[dst_t], add=True)

    plsc.subcore_barrier()
    for z in range(RPT // ZR):
        r0 = sid * RPT + z * ZR
        pltpu.sync_copy(acc.at[pl.ds(r0, ZR)], degp.at[cid, pl.ds(r0, ZR)])


_deg_call = pl.kernel(
    _deg_body,
    out_type=jax.ShapeDtypeStruct((NC, NP, D), jnp.float32),
    mesh=_mesh,
    scratch_types=[
        pltpu.VMEM((CH,), jnp.int32),
        pltpu.VMEM((TAIL,), jnp.int32),
        pltpu.VMEM((CH, D), jnp.float32),
        pltpu.VMEM((ZR, D), jnp.float32),
        pltpu.VMEM_SHARED((NP, D), jnp.float32),
    ],
)


def _agg_body(gp, src, dst, p, src_v, dst_v, src_t, dst_t, rows_v, rows_t, zbuf,
              acc, sem):
    cid = lax.axis_index("c")
    sid = lax.axis_index("s")
    wid = cid * NS + sid

    _zero_fill(zbuf, ZR, D)
    for z in range(RPT // ZR):
        pltpu.sync_copy(zbuf, acc.at[pl.ds(sid * RPT + z * ZR, ZR)])
    plsc.subcore_barrier()

    ebase = wid * EPW

    def chunk(s, _):
        base = ebase + s * CH
        pltpu.sync_copy(src.at[pl.ds(base, CH)], src_v)
        pltpu.sync_copy(dst.at[pl.ds(base, CH)], dst_v)
        pltpu.async_copy(gp.at[src_v], rows_v, sem).wait()
        pltpu.sync_copy(rows_v, acc.at[dst_v], add=True)
        return 0

    lax.fori_loop(0, NFULL, chunk, 0)
    base = ebase + NFULL * CH
    pltpu.sync_copy(src.at[pl.ds(base, TAIL)], src_t)
    pltpu.sync_copy(dst.at[pl.ds(base, TAIL)], dst_t)
    pltpu.async_copy(gp.at[src_t], rows_t, sem).wait()
    pltpu.sync_copy(rows_t, acc.at[dst_t], add=True)

    plsc.subcore_barrier()
    for z in range(RPT // ZR):
        r0 = sid * RPT + z * ZR
        pltpu.sync_copy(acc.at[pl.ds(r0, ZR)], p.at[cid, pl.ds(r0, ZR)])


_agg_call = pl.kernel(
    _agg_body,
    out_type=jax.ShapeDtypeStruct((NC, NP, D), jnp.float32),
    mesh=_mesh,
    scratch_types=[
        pltpu.VMEM((CH,), jnp.int32),
        pltpu.VMEM((CH,), jnp.int32),
        pltpu.VMEM((TAIL,), jnp.int32),
        pltpu.VMEM((TAIL,), jnp.int32),
        pltpu.VMEM((CH, D), jnp.float32),
        pltpu.VMEM((TAIL, D), jnp.float32),
        pltpu.VMEM((ZR, D), jnp.float32),
        pltpu.VMEM_SHARED((NP, D), jnp.float32),
        pltpu.SemaphoreType.DMA,
    ],
)

MB = 1000


def _k0_body(x_ref, w_ref, degp_ref, gp_ref, dinv_ref):
    d = degp_ref[...]
    deg = 1.0 + d[0, :, 0:1] + d[1, :, 0:1]
    dinv = lax.rsqrt(deg)
    g = jnp.dot(x_ref[...], w_ref[...], preferred_element_type=jnp.float32)
    gp_ref[...] = g * dinv
    dinv_ref[...] = dinv


def _k0_call(x, w, degp):
    return pl.pallas_call(
        _k0_body,
        grid=(N // MB,),
        in_specs=[
            pl.BlockSpec((MB, D), lambda i: (i, 0)),
            pl.BlockSpec((D, D), lambda i: (0, 0)),
            pl.BlockSpec((NC, MB, D), lambda i: (0, i, 0)),
        ],
        out_specs=[
            pl.BlockSpec((MB, D), lambda i: (i, 0)),
            pl.BlockSpec((MB, 1), lambda i: (i, 0)),
        ],
        out_shape=[
            jax.ShapeDtypeStruct((N, D), jnp.float32),
            jax.ShapeDtypeStruct((N, 1), jnp.float32),
        ],
    )(x, w, degp)


def _mid_body(p_ref, gp_ref, dinv_ref, b_ref, w_ref, out_ref):
    pr = p_ref[...]
    dinv = dinv_ref[...]
    h = dinv * (pr[0] + pr[1] + gp_ref[...]) + b_ref[...]
    h = jnp.maximum(h, 0.0)
    out_ref[...] = dinv * jnp.dot(h, w_ref[...],
                                  preferred_element_type=jnp.float32)


def _mid_call(p, gp, dinv, b2d, w):
    return pl.pallas_call(
        _mid_body,
        grid=(N // MB,),
        in_specs=[
            pl.BlockSpec((NC, MB, D), lambda i: (0, i, 0)),
            pl.BlockSpec((MB, D), lambda i: (i, 0)),
            pl.BlockSpec((MB, 1), lambda i: (i, 0)),
            pl.BlockSpec((1, D), lambda i: (0, 0)),
            pl.BlockSpec((D, D), lambda i: (0, 0)),
        ],
        out_specs=pl.BlockSpec((MB, D), lambda i: (i, 0)),
        out_shape=jax.ShapeDtypeStruct((N, D), jnp.float32),
    )(p, gp, dinv, b2d, w)


PB = 400
PG = N // PB


def _pool_body(p_ref, gp_ref, dinv_ref, b_ref, batch_ref, wout_ref, bout_ref,
               out_ref, gmax_s, gsum_s, cnt_s):
    i = pl.program_id(0)

    @pl.when(i == 0)
    def _init():
        gmax_s[...] = jnp.full((B, D), -jnp.inf, jnp.float32)
        gsum_s[...] = jnp.zeros((B, D), jnp.float32)
        cnt_s[...] = jnp.zeros((B, 1), jnp.float32)

    pr = p_ref[...]
    dinv = dinv_ref[...]
    h = dinv * (pr[0] + pr[1] + gp_ref[...]) + b_ref[...]
    h = jnp.maximum(h, 0.0)

    bbc = batch_ref[0]
    gids = lax.broadcasted_iota(jnp.int32, (PB, B), 1)
    onehot = (gids == bbc).astype(jnp.float32)
    gsum_s[...] += lax.dot_general(
        onehot, h, (((0,), (0,)), ((), ())),
        preferred_element_type=jnp.float32)
    cnt_s[...] += jnp.sum(onehot, axis=0)[:, None]

    lo = bbc[0, 0]
    hi = bbc[PB - 1, 0]

    def seg(g, _):
        mask = bbc == g
        m = jnp.max(jnp.where(mask, h, -jnp.inf), axis=0, keepdims=True)
        cur = gmax_s[pl.ds(g, 1), :]
        gmax_s[pl.ds(g, 1), :] = jnp.maximum(cur, m)
        return 0

    lax.fori_loop(lo, hi + 1, seg, 0)

    @pl.when(i == PG - 1)
    def _final():
        gmean = gsum_s[...] / jnp.maximum(cnt_s[...], 1.0)
        pooled = jnp.concatenate([gmax_s[...], gmean], axis=1)
        out_ref[...] = jnp.dot(pooled, wout_ref[...],
                               preferred_element_type=jnp.float32) + bout_ref[...]


def _pool_call(p, gp, dinv, b2d, batch3d, wout, bout2d, out_dim):
    return pl.pallas_call(
        _pool_body,
        grid=(PG,),
        in_specs=[
            pl.BlockSpec((NC, PB, D), lambda i: (0, i, 0)),
            pl.BlockSpec((PB, D), lambda i: (i, 0)),
            pl.BlockSpec((PB, 1), lambda i: (i, 0)),
            pl.BlockSpec((1, D), lambda i: (0, 0)),
            pl.BlockSpec((1, PB, 1), lambda i: (i, 0, 0)),
            pl.BlockSpec((2 * D, out_dim), lambda i: (0, 0)),
            pl.BlockSpec((1, out_dim), lambda i: (0, 0)),
        ],
        out_specs=pl.BlockSpec((B, out_dim), lambda i: (0, 0)),
        out_shape=jax.ShapeDtypeStruct((B, out_dim), jnp.float32),
        scratch_shapes=[
            pltpu.VMEM((B, D), jnp.float32),
            pltpu.VMEM((B, D), jnp.float32),
            pltpu.VMEM((B, 1), jnp.float32),
        ],
    )(p, gp, dinv, b2d, batch3d, wout, bout2d)


def kernel(x, edge_index, batch_index, W0, b0, W1, b1, W2, b2, W3, b3,
           Wout, bout):
    out_dim = Wout.shape[1]
    batch3d = batch_index.reshape(PG, PB, 1)

    src = edge_index[0]
    dst = edge_index[1]
    degp = _deg_call(dst)
    gp0, dinv = _k0_call(x, W0, degp)
    p0 = _agg_call(gp0, src, dst)
    gp1 = _mid_call(p0, gp0, dinv, b0.reshape(1, D), W1)
    p1 = _agg_call(gp1, src, dst)
    gp2 = _mid_call(p1, gp1, dinv, b1.reshape(1, D), W2)
    p2 = _agg_call(gp2, src, dst)
    gp3 = _mid_call(p2, gp2, dinv, b2.reshape(1, D), W3)
    p3 = _agg_call(gp3, src, dst)
    return _pool_call(p3, gp3, dinv, b3.reshape(1, D), batch3d, Wout,
                      bout.reshape(1, out_dim), out_dim)
